# Initial kernel scaffold; baseline (speedup 1.0000x reference)
import jax
import jax.numpy as jnp
from jax.experimental import pallas as pl


def kernel(z, codebook):
    raise NotImplementedError("write your pallas kernel here")


# speedup vs baseline: 6.2855x; 6.2855x over previous
import functools

import jax
import jax.numpy as jnp
from jax import lax
from jax.experimental import pallas as pl
from jax.experimental.pallas import tpu as pltpu
from jax.experimental.pallas import tpu_sc as plsc

N_E = 8192
E_DIM = 256
BETA = 0.25
N_TOK = 4608

T_BLK = 512
K_BLK = 2048
T_BLKS = N_TOK // T_BLK
K_BLKS = N_E // K_BLK

OH_T_BLK = 512
OH_K_BLK = 8192
OH_T_BLKS = N_TOK // OH_T_BLK
OH_K_BLKS = N_E // OH_K_BLK

SC_CORES = 2
SC_SUBCORES = 16
SC_WORKERS = SC_CORES * SC_SUBCORES
ROWS_PER_WORKER = N_TOK // SC_WORKERS


C_BLK = 256
N_CHUNK = K_BLK // C_BLK


def _argmin_body(z_ref, cb_ref, zsq_ref, esq_ref, idx_ref, loss_ref,
                 minv_ref, mini_ref):
    t = pl.program_id(0)
    k = pl.program_id(1)
    mm = lax.dot_general(z_ref[...], cb_ref[...],
                         (((1,), (1,)), ((), ())))
    d = (zsq_ref[...] + esq_ref[...]) - 2.0 * mm

    vmin = d[:, 0:C_BLK]
    varg = lax.broadcasted_iota(jnp.int32, (T_BLK, C_BLK), 1)
    for c in range(1, N_CHUNK):
        dc = d[:, c * C_BLK:(c + 1) * C_BLK]
        ic = lax.broadcasted_iota(jnp.int32, (T_BLK, C_BLK), 1) + c * C_BLK
        better = dc < vmin
        vmin = jnp.where(better, dc, vmin)
        varg = jnp.where(better, ic, varg)
    lmin = jnp.min(vmin, axis=1, keepdims=True)
    larg = jnp.min(jnp.where(vmin == lmin, varg, K_BLK), axis=1,
                   keepdims=True) + k * K_BLK

    @pl.when(k == 0)
    def _():
        minv_ref[...] = lmin
        mini_ref[...] = larg

    @pl.when(k > 0)
    def _():
        better = lmin < minv_ref[...]
        mini_ref[...] = jnp.where(better, larg, mini_ref[...])
        minv_ref[...] = jnp.where(better, lmin, minv_ref[...])

    @pl.when(k == K_BLKS - 1)
    def _():
        idx_ref[...] = mini_ref[...]

        @pl.when(t == 0)
        def _():
            loss_ref[0] = 0.0

        loss_ref[0] += jnp.sum(minv_ref[...])

        @pl.when(t == T_BLKS - 1)
        def _():
            loss_ref[0] = loss_ref[0] * ((1.0 + BETA) / (N_TOK * E_DIM))


def _onehot_body(idx_ref, out_ref):
    k = pl.program_id(1)
    iota = lax.broadcasted_iota(jnp.int32, (OH_T_BLK, OH_K_BLK), 1)
    out_ref[...] = (idx_ref[...] == iota + k * OH_K_BLK).astype(jnp.float32)


def _gather_body(cb_hbm, idx_hbm, zq_hbm, idx_v, rows_v, sem):
    wid = lax.axis_index("s") * SC_CORES + lax.axis_index("c")
    base = wid * ROWS_PER_WORKER
    pltpu.sync_copy(idx_hbm.at[pl.ds(base, ROWS_PER_WORKER)], idx_v)
    pltpu.async_copy(cb_hbm.at[idx_v], rows_v, sem).wait()
    pltpu.sync_copy(rows_v, zq_hbm.at[pl.ds(base, ROWS_PER_WORKER)])


def kernel(z, codebook):
    z32 = z.astype(jnp.float32)
    z_flat = z32.reshape(N_TOK, E_DIM)
    zsq = jnp.sum(z_flat ** 2, axis=1, keepdims=True)
    esq = jnp.sum(codebook ** 2, axis=1).reshape(1, N_E)

    idx, loss_v = pl.pallas_call(
        _argmin_body,
        grid=(T_BLKS, K_BLKS),
        in_specs=[
            pl.BlockSpec((T_BLK, E_DIM), lambda t, k: (t, 0)),
            pl.BlockSpec((K_BLK, E_DIM), lambda t, k: (k, 0)),
            pl.BlockSpec((T_BLK, 1), lambda t, k: (t, 0)),
            pl.BlockSpec((1, K_BLK), lambda t, k: (0, k)),
        ],
        out_specs=[
            pl.BlockSpec((T_BLK, 1), lambda t, k: (t, 0)),
            pl.BlockSpec(memory_space=pltpu.SMEM),
        ],
        out_shape=[
            jax.ShapeDtypeStruct((N_TOK, 1), jnp.int32),
            jax.ShapeDtypeStruct((1,), jnp.float32),
        ],
        scratch_shapes=[
            pltpu.VMEM((T_BLK, 1), jnp.float32),
            pltpu.VMEM((T_BLK, 1), jnp.int32),
        ],
        compiler_params=pltpu.CompilerParams(
            dimension_semantics=("arbitrary", "arbitrary")),
    )(z_flat, codebook, zsq, esq)

    min_encodings = pl.pallas_call(
        _onehot_body,
        grid=(OH_T_BLKS, OH_K_BLKS),
        in_specs=[pl.BlockSpec((OH_T_BLK, 1), lambda t, k: (t, 0))],
        out_specs=pl.BlockSpec((OH_T_BLK, OH_K_BLK), lambda t, k: (t, k)),
        out_shape=jax.ShapeDtypeStruct((N_TOK, N_E), jnp.float32),
        compiler_params=pltpu.CompilerParams(
            dimension_semantics=("arbitrary", "arbitrary")),
    )(idx)

    zq_flat = pl.kernel(
        _gather_body,
        out_type=jax.ShapeDtypeStruct((N_TOK, E_DIM), jnp.float32),
        mesh=plsc.VectorSubcoreMesh(core_axis_name="c", subcore_axis_name="s"),
        scratch_types=[
            pltpu.VMEM((ROWS_PER_WORKER,), jnp.int32),
            pltpu.VMEM((ROWS_PER_WORKER, E_DIM), jnp.float32),
            pltpu.SemaphoreType.DMA,
        ],
    )(codebook, idx.reshape(N_TOK))

    z_q = zq_flat.reshape(z32.shape)
    loss = loss_v[0]
    return (z_q, loss, min_encodings, idx.reshape(z.shape[0], -1))
